# Initial kernel scaffold; baseline (speedup 1.0000x reference)
import jax
import jax.numpy as jnp
from jax.experimental import pallas as pl


def kernel(x, x_embed_weight, batch_size):
    raise NotImplementedError("write your pallas kernel here")


# speedup vs baseline: 3.2279x; 3.2279x over previous
import functools

import jax
import jax.numpy as jnp
from jax import lax
from jax.experimental import pallas as pl
from jax.experimental.pallas import tpu as pltpu
from jax.experimental.pallas import tpu_sc as plsc

D = 256
CHUNK = 128


@functools.cache
def _build_gather(B):
    info = plsc.get_sparse_core_info()
    n_workers = info.num_cores * info.num_subcores
    per_w = B // n_workers
    n_chunks = per_w // CHUNK
    mesh = plsc.VectorSubcoreMesh(core_axis_name="c", subcore_axis_name="s")

    @functools.partial(
        pl.kernel,
        mesh=mesh,
        out_type=jax.ShapeDtypeStruct((B, D), jnp.float32),
        scratch_types=[
            pltpu.VMEM((CHUNK,), jnp.int32),
            pltpu.VMEM((CHUNK,), jnp.int32),
            pltpu.VMEM((CHUNK, D), jnp.float32),
            pltpu.VMEM((CHUNK, D), jnp.float32),
            pltpu.SemaphoreType.DMA,
            pltpu.SemaphoreType.DMA,
        ],
    )
    def gather_kernel(table_hbm, idx_hbm, out_hbm, idx0, idx1, rows0, rows1,
                      sem0, sem1):
        wid = lax.axis_index("s") * info.num_cores + lax.axis_index("c")
        base = wid * per_w
        idx_bufs = (idx0, idx1)
        row_bufs = (rows0, rows1)
        sems = (sem0, sem1)

        copies = [None] * n_chunks
        pltpu.sync_copy(idx_hbm.at[pl.ds(base, CHUNK)], idx0)
        copies[0] = pltpu.async_copy(table_hbm.at[idx0], rows0, sem0)
        for i in range(n_chunks):
            cur = i % 2
            nxt = 1 - cur
            if i + 1 < n_chunks:
                off = base + (i + 1) * CHUNK
                pltpu.sync_copy(idx_hbm.at[pl.ds(off, CHUNK)], idx_bufs[nxt])
                copies[i + 1] = pltpu.async_copy(
                    table_hbm.at[idx_bufs[nxt]], row_bufs[nxt], sems[nxt])
            copies[i].wait()
            pltpu.sync_copy(row_bufs[cur],
                            out_hbm.at[pl.ds(base + i * CHUNK, CHUNK)])

    return gather_kernel


def kernel(x, x_embed_weight, batch_size=1):
    b, t = x.shape
    flat = x.reshape(b * t).astype(jnp.int32)
    out = _build_gather(b * t)(x_embed_weight, flat)
    return out.reshape(b, t, D)


# speedup vs baseline: 3.2400x; 1.0037x over previous
import functools

import jax
import jax.numpy as jnp
from jax import lax
from jax.experimental import pallas as pl
from jax.experimental.pallas import tpu as pltpu
from jax.experimental.pallas import tpu_sc as plsc

D = 256
CHUNK = 128
NBUF = 3


@functools.cache
def _build_gather(B):
    info = plsc.get_sparse_core_info()
    n_workers = info.num_cores * info.num_subcores
    per_w = B // n_workers
    n_chunks = per_w // CHUNK
    mesh = plsc.VectorSubcoreMesh(core_axis_name="c", subcore_axis_name="s")

    @functools.partial(
        pl.kernel,
        mesh=mesh,
        out_type=jax.ShapeDtypeStruct((B, D), jnp.float32),
        scratch_types=[
            pltpu.VMEM((per_w,), jnp.int32),
            pltpu.VMEM((NBUF, CHUNK, D), jnp.float32),
            pltpu.SemaphoreType.DMA((NBUF,)),
            pltpu.SemaphoreType.DMA((NBUF,)),
        ],
    )
    def gather_kernel(table_hbm, idx_hbm, out_hbm, idx_v, rows_v, gsem, wsem):
        wid = lax.axis_index("s") * info.num_cores + lax.axis_index("c")
        base = wid * per_w
        pltpu.sync_copy(idx_hbm.at[pl.ds(base, per_w)], idx_v)

        def gather(i):
            return pltpu.async_copy(
                table_hbm.at[idx_v.at[pl.ds(i * CHUNK, CHUNK)]],
                rows_v.at[i % NBUF], gsem.at[i % NBUF])

        def write(i):
            return pltpu.async_copy(
                rows_v.at[i % NBUF],
                out_hbm.at[pl.ds(base + i * CHUNK, CHUNK)],
                wsem.at[i % NBUF])

        gathers = [None] * n_chunks
        writes = [None] * n_chunks
        for i in range(min(NBUF - 1, n_chunks)):
            gathers[i] = gather(i)
        for i in range(n_chunks):
            gathers[i].wait()
            writes[i] = write(i)
            nxt = i + NBUF - 1
            if nxt < n_chunks:
                if i >= 1:
                    writes[i - 1].wait()
                gathers[nxt] = gather(nxt)
        for i in range(max(0, n_chunks - NBUF), n_chunks):
            writes[i].wait()

    return gather_kernel


def kernel(x, x_embed_weight, batch_size=1):
    b, t = x.shape
    flat = x.reshape(b * t).astype(jnp.int32)
    out = _build_gather(b * t)(x_embed_weight, flat)
    return out.reshape(b, t, D)


# speedup vs baseline: 3.2472x; 1.0022x over previous
import functools

import jax
import jax.numpy as jnp
from jax import lax
from jax.experimental import pallas as pl
from jax.experimental.pallas import tpu as pltpu
from jax.experimental.pallas import tpu_sc as plsc

D = 256
CHUNK = 128
NBUF = 3


@functools.cache
def _build_gather(nb, nt):
    info = plsc.get_sparse_core_info()
    n_workers = info.num_cores * info.num_subcores
    per_w = (nb * nt) // n_workers
    w_per_row = nt // per_w
    n_chunks = per_w // CHUNK
    mesh = plsc.VectorSubcoreMesh(core_axis_name="c", subcore_axis_name="s")

    @functools.partial(
        pl.kernel,
        mesh=mesh,
        out_type=jax.ShapeDtypeStruct((nb, nt, D), jnp.float32),
        scratch_types=[
            pltpu.VMEM((per_w,), jnp.int32),
            pltpu.VMEM((NBUF, CHUNK, D), jnp.float32),
            pltpu.SemaphoreType.DMA((NBUF,)),
            pltpu.SemaphoreType.DMA((NBUF,)),
        ],
    )
    def gather_kernel(table_hbm, idx_hbm, out_hbm, idx_v, rows_v, gsem, wsem):
        wid = lax.axis_index("s") * info.num_cores + lax.axis_index("c")
        row = wid // w_per_row
        col = (wid % w_per_row) * per_w
        pltpu.sync_copy(idx_hbm.at[row, pl.ds(col, per_w)], idx_v)

        def gather(i):
            return pltpu.async_copy(
                table_hbm.at[idx_v.at[pl.ds(i * CHUNK, CHUNK)]],
                rows_v.at[i % NBUF], gsem.at[i % NBUF])

        def write(i):
            return pltpu.async_copy(
                rows_v.at[i % NBUF],
                out_hbm.at[row, pl.ds(col + i * CHUNK, CHUNK)],
                wsem.at[i % NBUF])

        gathers = [None] * n_chunks
        writes = [None] * n_chunks
        for i in range(min(NBUF - 1, n_chunks)):
            gathers[i] = gather(i)
        for i in range(n_chunks):
            gathers[i].wait()
            writes[i] = write(i)
            nxt = i + NBUF - 1
            if nxt < n_chunks:
                if i >= 1:
                    writes[i - 1].wait()
                gathers[nxt] = gather(nxt)
        for i in range(max(0, n_chunks - NBUF), n_chunks):
            writes[i].wait()

    return gather_kernel


def kernel(x, x_embed_weight, batch_size=1):
    nb, nt = x.shape
    return _build_gather(nb, nt)(x_embed_weight, x)
